# baseline (speedup 1.0000x reference)
import functools

import jax
import jax.numpy as jnp
from jax import lax
from jax.experimental import pallas as pl
from jax.experimental.pallas import tpu as pltpu
from jax.experimental.pallas import tpu_sc as plsc

B = 16384
DD = 13
DS = 26
NF = DD + DS
H = 32
L = 16

NW = 32
BPW = B // NW
CBL = 32
CHUNKS = BPW // CBL
NBLK = CBL // L


def _scalar(ref, r, c):
    row = ref[r, pl.ds((c // L) * L, L)]
    return row[c % L]


def _body(x_hbm, idx_hbm, w_hbm, table_hbm, col_hbm, out_hbm,
          idx_v, x_v, gath_v, sout_v, dout_v, w_v, col_v,
          sem_g, sem_o):
    wid = lax.axis_index("s") * 2 + lax.axis_index("c")

    pltpu.sync_copy(w_hbm, w_v)
    pltpu.sync_copy(col_hbm, col_v)
    iota = jax.lax.iota(jnp.int32, L)

    @pl.loop(0, CHUNKS)
    def _chunk(ci):
        p = ci % 2
        c0 = pl.multiple_of(wid * BPW + ci * CBL, 8)

        pltpu.sync_copy(idx_hbm.at[:, pl.ds(c0, CBL)], idx_v)
        pltpu.sync_copy(x_hbm.at[:, pl.ds(c0, CBL)], x_v)

        for s in range(DS):
            pltpu.async_copy(
                table_hbm.at[idx_v.at[s]],
                gath_v.at[pl.ds(s * CBL, CBL)],
                sem_g,
            )

        @pl.when(ci >= 2)
        def _():
            pltpu.make_async_copy(
                dout_v.at[0], out_hbm.at[pl.ds(0, DD), :, pl.ds(0, CBL)],
                sem_o).wait()
            pltpu.make_async_copy(
                sout_v.at[0], out_hbm.at[pl.ds(DD, DS), :, pl.ds(0, CBL)],
                sem_o).wait()

        @pl.loop(0, DD)
        def _dense(f):
            for hh in range(0, H, L):
                wrow = w_v[f, pl.ds(hh, L)]
                crow = col_v[f, pl.ds(hh, L)]
                for j in range(L):
                    h = hh + j
                    w = wrow[j]
                    c = crow[j]
                    for blk in range(NBLK):
                        xb = x_v[f, pl.ds(blk * L, L)]
                        dout_v[p, f, h, pl.ds(blk * L, L)] = (
                            jnp.maximum(xb * w, 0.0) + c)

        pltpu.async_copy(
            dout_v.at[p],
            out_hbm.at[pl.ds(0, DD), :, pl.ds(c0, CBL)],
            sem_o,
        )

        for s in range(DS):
            pltpu.make_async_copy(
                table_hbm.at[idx_v.at[s]],
                gath_v.at[pl.ds(s * CBL, CBL)],
                sem_g,
            ).wait()

        @pl.loop(0, DS)
        def _sparse(s):
            base = s * CBL
            for hh in range(0, H, L):
                crow = col_v[DD + s, pl.ds(hh, L)]
                for j in range(L):
                    h = hh + j
                    c = crow[j]
                    hvec = jnp.full((L,), h, dtype=jnp.int32)
                    for blk in range(NBLK):
                        rows = iota + (base + blk * L)
                        g = plsc.load_gather(gath_v, [rows, hvec])
                        sout_v[p, s, h, pl.ds(blk * L, L)] = g + c

        pltpu.async_copy(
            sout_v.at[p],
            out_hbm.at[pl.ds(DD, DS), :, pl.ds(c0, CBL)],
            sem_o,
        )

    for _ in range(2):
        pltpu.make_async_copy(
            dout_v.at[0], out_hbm.at[pl.ds(0, DD), :, pl.ds(0, CBL)],
            sem_o).wait()
        pltpu.make_async_copy(
            sout_v.at[0], out_hbm.at[pl.ds(DD, DS), :, pl.ds(0, CBL)],
            sem_o).wait()


TBC = 512
TBO = TBC // 4
TV = 1000000


def _xpose_body(in_ref, out_ref):
    x = in_ref[...]
    c = jax.lax.broadcasted_iota(jnp.int32, (TBC, TBO), 0)
    p = jax.lax.broadcasted_iota(jnp.int32, (TBC, TBO), 1)
    for k in range(4):
        sel = (c == 4 * p + k).astype(jnp.float32)
        r = jax.lax.dot_general(sel, x, (((0,), (1,)), ((), ())),
                                preferred_element_type=jnp.float32)
        out_ref[:, 32 * k:32 * (k + 1)] = r


@functools.cache
def _tc_xpose():
    return pl.pallas_call(
        _xpose_body,
        grid=((TV + TBC - 1) // TBC,),
        in_specs=[pl.BlockSpec((H, TBC), lambda j: (0, j))],
        out_specs=pl.BlockSpec((TBO, 128), lambda j: (j, 0)),
        out_shape=jax.ShapeDtypeStruct((TV // 4, 128), jnp.float32),
    )


@functools.cache
def _sc_embed():
    mesh = plsc.VectorSubcoreMesh(core_axis_name="c", subcore_axis_name="s")
    return functools.partial(
        pl.kernel,
        out_type=jax.ShapeDtypeStruct((NF, H, B), jnp.float32),
        mesh=mesh,
        scratch_types=[
            pltpu.VMEM((DS, CBL), jnp.int32),
            pltpu.VMEM((DD, CBL), jnp.float32),
            pltpu.VMEM((DS * CBL, H), jnp.float32),
            pltpu.VMEM((2, DS, H, CBL), jnp.float32),
            pltpu.VMEM((2, DD, H, CBL), jnp.float32),
            pltpu.VMEM((DD, H), jnp.float32),
            pltpu.VMEM((NF, H), jnp.float32),
            pltpu.SemaphoreType.DMA,
            pltpu.SemaphoreType.DMA,
        ],
        compiler_params=pltpu.CompilerParams(
            use_tc_tiling_on_sc=False, needs_layout_passes=False),
    )(_body)


def kernel(x_dense, x_sparse, dense_embed_weight, sparse_embed_weight,
           col_embed):
    xt = x_dense.T
    idxt = x_sparse.astype(jnp.int32).T
    tbl = _tc_xpose()(sparse_embed_weight.T).reshape(TV, H)
    out = _sc_embed()(xt, idxt, dense_embed_weight, tbl, col_embed)
    return jnp.transpose(out, (2, 0, 1))


# speedup vs baseline: 1.9418x; 1.9418x over previous
import functools

import jax
import jax.numpy as jnp
from jax import lax
from jax.experimental import pallas as pl
from jax.experimental.pallas import tpu as pltpu
from jax.experimental.pallas import tpu_sc as plsc

B = 16384
DD = 13
DS = 26
NF = DD + DS
H = 32
L = 16

NW = 32
BPW = B // NW
CBL = 16
CHUNKS = BPW // CBL


def _fire_gathers(table_hbm, idx_v, gath_v, q, sem_g):
    for s in range(DS):
        pltpu.async_copy(
            table_hbm.at[idx_v.at[q, s]],
            gath_v.at[q, pl.ds(s * CBL, CBL), :],
            sem_g,
        )


def _drain_gathers(table_hbm, idx_v, gath_v, q, sem_g):
    for s in range(DS):
        pltpu.make_async_copy(
            table_hbm.at[idx_v.at[q, s]],
            gath_v.at[q, pl.ds(s * CBL, CBL), :],
            sem_g,
        ).wait()


def _body(x_hbm, idx_hbm, w_hbm, table_hbm, col_hbm, out_hbm,
          idx_v, x_v, gath_v, sout_v, dout_v, w_v, col_v,
          sem_g, sem_in, sem_o):
    wid = lax.axis_index("s") * 2 + lax.axis_index("c")
    base = wid * BPW

    pltpu.sync_copy(w_hbm, w_v)
    pltpu.sync_copy(col_hbm, col_v)
    iota = jax.lax.iota(jnp.int32, L)

    def fire_in(ci, q):
        c = pl.multiple_of(base + ci * CBL, 8)
        pltpu.async_copy(idx_hbm.at[:, pl.ds(c, CBL)], idx_v.at[q], sem_in)
        pltpu.async_copy(x_hbm.at[:, pl.ds(c, CBL)], x_v.at[q], sem_in)

    def wait_in(ci, q):
        c = pl.multiple_of(base + ci * CBL, 8)
        pltpu.make_async_copy(
            idx_hbm.at[:, pl.ds(c, CBL)], idx_v.at[q], sem_in).wait()
        pltpu.make_async_copy(
            x_hbm.at[:, pl.ds(c, CBL)], x_v.at[q], sem_in).wait()

    def wait_out(pp):
        pltpu.make_async_copy(
            dout_v.at[pp], out_hbm.at[pl.ds(0, DD), :, pl.ds(0, CBL)],
            sem_o).wait()
        pltpu.make_async_copy(
            sout_v.at[pp], out_hbm.at[pl.ds(DD, DS), :, pl.ds(0, CBL)],
            sem_o).wait()

    pltpu.sync_copy(idx_hbm.at[:, pl.ds(pl.multiple_of(base, 8), CBL)],
                    idx_v.at[0])
    pltpu.sync_copy(x_hbm.at[:, pl.ds(pl.multiple_of(base, 8), CBL)],
                    x_v.at[0])
    _fire_gathers(table_hbm, idx_v, gath_v, 0, sem_g)
    fire_in(1, 1)

    @pl.loop(0, CHUNKS)
    def _chunk(ci):
        p = ci % 2
        c0 = pl.multiple_of(base + ci * CBL, 8)

        _drain_gathers(table_hbm, idx_v, gath_v, p, sem_g)

        @pl.when(ci + 1 < CHUNKS)
        def _():
            wait_in(ci + 1, 1 - p)
            _fire_gathers(table_hbm, idx_v, gath_v, 1 - p, sem_g)

        @pl.when(ci >= 2)
        def _():
            wait_out(p)

        @pl.loop(0, DD)
        def _dense(f):
            xb = x_v[p, f, pl.ds(0, L)]
            for hh in range(0, H, L):
                wrow = w_v[f, pl.ds(hh, L)]
                crow = col_v[f, pl.ds(hh, L)]
                for j in range(L):
                    dout_v[p, f, hh + j, pl.ds(0, L)] = (
                        jnp.maximum(xb * wrow[j], 0.0) + crow[j])

        pltpu.async_copy(
            dout_v.at[p],
            out_hbm.at[pl.ds(0, DD), :, pl.ds(c0, CBL)],
            sem_o,
        )

        @pl.when(ci + 2 < CHUNKS)
        def _():
            fire_in(ci + 2, p)

        @pl.loop(0, DS)
        def _sparse(s):
            rows = iota + s * CBL
            for hh in range(0, H, L):
                crow = col_v[DD + s, pl.ds(hh, L)]
                for j in range(L):
                    h = hh + j
                    hvec = jnp.full((L,), h, dtype=jnp.int32)
                    g = plsc.load_gather(gath_v.at[p], [rows, hvec])
                    sout_v[p, s, h, pl.ds(0, L)] = g + crow[j]

        pltpu.async_copy(
            sout_v.at[p],
            out_hbm.at[pl.ds(DD, DS), :, pl.ds(c0, CBL)],
            sem_o,
        )

    wait_out(0)
    wait_out(1)


@functools.cache
def _sc_embed():
    mesh = plsc.VectorSubcoreMesh(core_axis_name="c", subcore_axis_name="s")
    return functools.partial(
        pl.kernel,
        out_type=jax.ShapeDtypeStruct((NF, H, B), jnp.float32),
        mesh=mesh,
        scratch_types=[
            pltpu.VMEM((2, DS, CBL), jnp.int32),
            pltpu.VMEM((2, DD, CBL), jnp.float32),
            pltpu.VMEM((2, DS * CBL, H), jnp.float32),
            pltpu.VMEM((2, DS, H, CBL), jnp.float32),
            pltpu.VMEM((2, DD, H, CBL), jnp.float32),
            pltpu.VMEM((DD, H), jnp.float32),
            pltpu.VMEM((NF, H), jnp.float32),
            pltpu.SemaphoreType.DMA,
            pltpu.SemaphoreType.DMA,
            pltpu.SemaphoreType.DMA,
        ],
        compiler_params=pltpu.CompilerParams(
            use_tc_tiling_on_sc=False, needs_layout_passes=False),
    )(_body)


def kernel(x_dense, x_sparse, dense_embed_weight, sparse_embed_weight,
           col_embed):
    xt = x_dense.T
    idxt = x_sparse.astype(jnp.int32).T
    out = _sc_embed()(xt, idxt, dense_embed_weight, sparse_embed_weight,
                      col_embed)
    return jnp.transpose(out, (2, 0, 1))


# speedup vs baseline: 1.9692x; 1.0141x over previous
import functools

import jax
import jax.numpy as jnp
from jax import lax
from jax.experimental import pallas as pl
from jax.experimental.pallas import tpu as pltpu
from jax.experimental.pallas import tpu_sc as plsc

B = 16384
DD = 13
DS = 26
NF = DD + DS
H = 32
L = 16

NW = 32
BPW = B // NW
CBL = 128
CHUNKS = BPW // CBL
NBLK = CBL // L
TV = 1000000
TP = TV // 4


def _body(x_hbm, idx_hbm, w_hbm, tbl_hbm, col_hbm, out_hbm,
          idx_v, qidx_v, x_v, gath_v, plane_v, df_v, w_v, col_v,
          sem_g, sem_od, sem_os):
    wid = lax.axis_index("s") * 2 + lax.axis_index("c")
    base = wid * BPW

    pltpu.sync_copy(w_hbm, w_v)
    pltpu.sync_copy(col_hbm, col_v)
    iota = jax.lax.iota(jnp.int32, L)
    iota128 = iota * 128

    def fire_gather(s):
        pltpu.async_copy(tbl_hbm.at[qidx_v.at[s]], gath_v.at[s % 2], sem_g)

    def drain_gather(s):
        pltpu.make_async_copy(
            tbl_hbm.at[qidx_v.at[s]], gath_v.at[s % 2], sem_g).wait()

    @pl.loop(0, CHUNKS)
    def _chunk(ci):
        c0 = pl.multiple_of(base + ci * CBL, 128)

        pltpu.sync_copy(idx_hbm.at[:, pl.ds(c0, CBL)], idx_v)
        pltpu.sync_copy(x_hbm.at[:, pl.ds(c0, CBL)], x_v)

        @pl.loop(0, DS)
        def _q(s):
            for blk in range(NBLK):
                v = idx_v[s, pl.ds(blk * L, L)]
                qidx_v[s, pl.ds(blk * L, L)] = (
                    lax.shift_right_logical(v, 2))

        fire_gather(0)
        fire_gather(1)

        @pl.loop(0, DD)
        def _dense(f):
            fp = f % 2

            @pl.when(f >= 2)
            def _():
                pltpu.make_async_copy(
                    df_v.at[0], out_hbm.at[0, :, pl.ds(0, CBL)],
                    sem_od).wait()

            for hh in range(0, H, L):
                wrow = w_v[f, pl.ds(hh, L)]
                crow = col_v[f, pl.ds(hh, L)]
                for j in range(L):
                    w = wrow[j]
                    c = crow[j]
                    for blk in range(NBLK):
                        xb = x_v[f, pl.ds(blk * L, L)]
                        df_v[fp, hh + j, pl.ds(blk * L, L)] = (
                            jnp.maximum(xb * w, 0.0) + c)

            pltpu.async_copy(
                df_v.at[fp], out_hbm.at[f, :, pl.ds(c0, CBL)], sem_od)

        @pl.loop(0, DS)
        def _sparse(s):
            par = s % 2
            drain_gather(s)

            @pl.when(s >= 2)
            def _():
                pltpu.make_async_copy(
                    plane_v.at[0], out_hbm.at[DD, :, pl.ds(0, CBL)],
                    sem_os).wait()

            gv = gath_v.at[par]
            for blk in range(NBLK):
                idxb = idx_v[s, pl.ds(blk * L, L)]
                colv0 = (idxb & 3) << 5
                rows = iota + blk * L
                for hh in range(0, H, L):
                    crow = col_v[DD + s, pl.ds(hh, L)]
                    for j in range(L):
                        g = plsc.load_gather(gv, [rows, colv0 + (hh + j)])
                        plane_v[par, hh + j, pl.ds(blk * L, L)] = g + crow[j]

            @pl.when(s + 2 < DS)
            def _():
                fire_gather(s + 2)

            pltpu.async_copy(
                plane_v.at[par], out_hbm.at[DD + s, :, pl.ds(c0, CBL)],
                sem_os)

        for _ in range(2):
            pltpu.make_async_copy(
                df_v.at[0], out_hbm.at[0, :, pl.ds(0, CBL)], sem_od).wait()
            pltpu.make_async_copy(
                plane_v.at[0], out_hbm.at[DD, :, pl.ds(0, CBL)],
                sem_os).wait()


@functools.cache
def _sc_embed():
    mesh = plsc.VectorSubcoreMesh(core_axis_name="c", subcore_axis_name="s")
    return functools.partial(
        pl.kernel,
        out_type=jax.ShapeDtypeStruct((NF, H, B), jnp.float32),
        mesh=mesh,
        scratch_types=[
            pltpu.VMEM((DS, CBL), jnp.int32),
            pltpu.VMEM((DS, CBL), jnp.int32),
            pltpu.VMEM((DD, CBL), jnp.float32),
            pltpu.VMEM((2, CBL, 128), jnp.float32),
            pltpu.VMEM((2, H, CBL), jnp.float32),
            pltpu.VMEM((2, H, CBL), jnp.float32),
            pltpu.VMEM((DD, H), jnp.float32),
            pltpu.VMEM((NF, H), jnp.float32),
            pltpu.SemaphoreType.DMA,
            pltpu.SemaphoreType.DMA,
            pltpu.SemaphoreType.DMA,
        ],
        compiler_params=pltpu.CompilerParams(
            use_tc_tiling_on_sc=True, needs_layout_passes=False),
    )(_body)


def kernel(x_dense, x_sparse, dense_embed_weight, sparse_embed_weight,
           col_embed):
    xt = x_dense.T
    idxt = x_sparse.astype(jnp.int32).T
    tbl4 = sparse_embed_weight.reshape(TP, 128)
    out = _sc_embed()(xt, idxt, dense_embed_weight, tbl4, col_embed)
    return jnp.transpose(out, (2, 0, 1))


# speedup vs baseline: 2.0078x; 1.0196x over previous
import functools

import jax
import jax.numpy as jnp
from jax import lax
from jax.experimental import pallas as pl
from jax.experimental.pallas import tpu as pltpu
from jax.experimental.pallas import tpu_sc as plsc

B = 16384
DD = 13
DS = 26
NF = DD + DS
H = 32
L = 16

NW = 32
BPW = B // NW
CBL = 128
CHUNKS = BPW // CBL
NBLK = CBL // L
TV = 1000000
TP = TV // 4


def _body(x_hbm, idx_hbm, w_hbm, tbl_hbm, col_hbm, out_hbm,
          idx_v, x_v, gath_v, plane_v, df_v, w_v, col_v,
          sem_g, sem_od, sem_os):
    wid = lax.axis_index("s") * 2 + lax.axis_index("c")
    base = wid * BPW

    pltpu.sync_copy(w_hbm, w_v)
    pltpu.sync_copy(col_hbm, col_v)
    iota = jax.lax.iota(jnp.int32, L)
    iota128 = iota * 128

    def fire_gather(s):
        pltpu.async_copy(tbl_hbm.at[idx_v.at[s]], gath_v.at[s % 2], sem_g)

    def drain_gather(s):
        pltpu.make_async_copy(
            tbl_hbm.at[idx_v.at[s]], gath_v.at[s % 2], sem_g).wait()

    @pl.loop(0, CHUNKS)
    def _chunk(ci):
        c0 = pl.multiple_of(base + ci * CBL, 128)

        pltpu.sync_copy(idx_hbm.at[:, pl.ds(c0, CBL)], idx_v)
        pltpu.sync_copy(x_hbm.at[:, pl.ds(c0, CBL)], x_v)

        fire_gather(0)
        fire_gather(1)

        @pl.loop(0, DD)
        def _dense(f):
            fp = f % 2

            @pl.when(f >= 2)
            def _():
                pltpu.make_async_copy(
                    df_v.at[0], out_hbm.at[0, :, pl.ds(0, CBL)],
                    sem_od).wait()

            for hh in range(0, H, L):
                wrow = w_v[f, pl.ds(hh, L)]
                crow = col_v[f, pl.ds(hh, L)]
                for j in range(L):
                    w = wrow[j]
                    c = crow[j]
                    for blk in range(NBLK):
                        xb = x_v[f, pl.ds(blk * L, L)]
                        df_v[fp, hh + j, pl.ds(blk * L, L)] = (
                            jnp.maximum(xb * w, 0.0) + c)

            pltpu.async_copy(
                df_v.at[fp], out_hbm.at[f, :, pl.ds(c0, CBL)], sem_od)

        @pl.loop(0, DS)
        def _sparse(s):
            par = s % 2
            drain_gather(s)

            @pl.when(s >= 2)
            def _():
                pltpu.make_async_copy(
                    plane_v.at[0], out_hbm.at[DD, :, pl.ds(0, CBL)],
                    sem_os).wait()

            gv = gath_v.at[par]
            for blk in range(NBLK):
                rows = iota + blk * L
                for hh in range(0, H, L):
                    crow = col_v[DD + s, pl.ds(hh, L)]
                    for j in range(L):
                        hvec = jnp.full((L,), hh + j, dtype=jnp.int32)
                        g = plsc.load_gather(gv, [rows, hvec])
                        plane_v[par, hh + j, pl.ds(blk * L, L)] = g + crow[j]

            @pl.when(s + 2 < DS)
            def _():
                fire_gather(s + 2)

            pltpu.async_copy(
                plane_v.at[par], out_hbm.at[DD + s, :, pl.ds(c0, CBL)],
                sem_os)

        for _ in range(2):
            pltpu.make_async_copy(
                df_v.at[0], out_hbm.at[0, :, pl.ds(0, CBL)], sem_od).wait()
            pltpu.make_async_copy(
                plane_v.at[0], out_hbm.at[DD, :, pl.ds(0, CBL)],
                sem_os).wait()


@functools.cache
def _sc_embed():
    mesh = plsc.VectorSubcoreMesh(core_axis_name="c", subcore_axis_name="s")
    return functools.partial(
        pl.kernel,
        out_type=jax.ShapeDtypeStruct((NF, H, B), jnp.float32),
        mesh=mesh,
        scratch_types=[
            pltpu.VMEM((DS, CBL), jnp.int32),
            pltpu.VMEM((DD, CBL), jnp.float32),
            pltpu.VMEM((2, CBL, 128), jnp.float32),
            pltpu.VMEM((2, H, CBL), jnp.float32),
            pltpu.VMEM((2, H, CBL), jnp.float32),
            pltpu.VMEM((DD, H), jnp.float32),
            pltpu.VMEM((NF, H), jnp.float32),
            pltpu.SemaphoreType.DMA,
            pltpu.SemaphoreType.DMA,
            pltpu.SemaphoreType.DMA,
        ],
        compiler_params=pltpu.CompilerParams(
            use_tc_tiling_on_sc=True, needs_layout_passes=False),
    )(_body)


def kernel(x_dense, x_sparse, dense_embed_weight, sparse_embed_weight,
           col_embed):
    xt = x_dense.T
    idxt = x_sparse.astype(jnp.int32).T
    tblp = jnp.pad(sparse_embed_weight, ((0, 0), (0, 128 - H)))
    out = _sc_embed()(xt, idxt, dense_embed_weight, tblp, col_embed)
    return jnp.transpose(out, (2, 0, 1))
